# baseline (speedup 1.0000x reference)
import jax
import jax.numpy as jnp
from jax.experimental import pallas as pl
from jax.experimental.pallas import tpu as pltpu

_D_IN = 4096
_D_H = 2048
_BLK = 256


def _body(ab_ref, w_ref, ws_ref, be_ref, bs_ref, out_ref):
    i = pl.program_id(0)
    e = jnp.dot(ab_ref[...], w_ref[...], preferred_element_type=jnp.float32)
    part = jnp.sum(e * ws_ref[...])

    @pl.when(i == 0)
    def _():
        bias = bs_ref[0, 0] + jnp.sum(be_ref[...] * (ws_ref[0:1, :] + ws_ref[1:2, :]))
        out_ref[0, 0] = bias

    out_ref[0, 0] += part


def kernel(a, b, W_embed, b_embed, W_scorer, b_scorer):
    ab = jnp.stack([a, b])
    ws = W_scorer.reshape(2, _D_H)
    be = b_embed.reshape(1, _D_H)
    bs = b_scorer.reshape(1, 1)
    out = pl.pallas_call(
        _body,
        grid=(_D_IN // _BLK,),
        in_specs=[
            pl.BlockSpec((2, _BLK), lambda i: (0, i)),
            pl.BlockSpec((_BLK, _D_H), lambda i: (i, 0)),
            pl.BlockSpec((2, _D_H), lambda i: (0, 0)),
            pl.BlockSpec((1, _D_H), lambda i: (0, 0)),
            pl.BlockSpec(memory_space=pltpu.SMEM),
        ],
        out_specs=pl.BlockSpec(memory_space=pltpu.SMEM),
        out_shape=jax.ShapeDtypeStruct((1, 1), jnp.float32),
    )(ab, W_embed, ws, be, bs)
    return out.reshape(())


# speedup vs baseline: 1.3371x; 1.3371x over previous
import jax
import jax.numpy as jnp
from jax.experimental import pallas as pl
from jax.experimental.pallas import tpu as pltpu

_D_IN = 4096
_D_H = 2048
_BLK = 1024


def _body(ab_ref, w_ref, ws_ref, be_ref, bs_ref, out_ref):
    i = pl.program_id(0)
    e = jnp.dot(ab_ref[...], w_ref[...], preferred_element_type=jnp.float32)
    part = jnp.sum(e * ws_ref[...])

    @pl.when(i == 0)
    def _():
        bias = bs_ref[0, 0] + jnp.sum(be_ref[...] * (ws_ref[0:1, :] + ws_ref[1:2, :]))
        out_ref[0, 0] = bias

    out_ref[0, 0] += part


def kernel(a, b, W_embed, b_embed, W_scorer, b_scorer):
    ab = jnp.stack([a, b])
    ws = W_scorer.reshape(2, _D_H)
    be = b_embed.reshape(1, _D_H)
    bs = b_scorer.reshape(1, 1)
    out = pl.pallas_call(
        _body,
        grid=(_D_IN // _BLK,),
        in_specs=[
            pl.BlockSpec((2, _BLK), lambda i: (0, i)),
            pl.BlockSpec((_BLK, _D_H), lambda i: (i, 0)),
            pl.BlockSpec((2, _D_H), lambda i: (0, 0)),
            pl.BlockSpec((1, _D_H), lambda i: (0, 0)),
            pl.BlockSpec(memory_space=pltpu.SMEM),
        ],
        out_specs=pl.BlockSpec(memory_space=pltpu.SMEM),
        out_shape=jax.ShapeDtypeStruct((1, 1), jnp.float32),
    )(ab, W_embed, ws, be, bs)
    return out.reshape(())
